# Initial kernel scaffold; baseline (speedup 1.0000x reference)
import jax
import jax.numpy as jnp
from jax.experimental import pallas as pl


def kernel(x, edge_index, edge_attr, fc1_W, fc1_b, k1_W, k1_b, k2_W, k2_b, root_W, conv_b, fc2_W, fc2_b):
    raise NotImplementedError("write your pallas kernel here")


# speedup vs baseline: 2.2036x; 2.2036x over previous
import functools

import jax
import jax.numpy as jnp
from jax import lax
from jax.experimental import pallas as pl
from jax.experimental.pallas import tpu as pltpu
from jax.experimental.pallas import tpu_sc as plsc

N = 10000
E = 160000
W = 32
DEPTH = 4
R = 9
NC, NS = 2, 16
NW = NC * NS
CH = 128
NCHUNK = E // CH
CPW = 40
NCHUNKP = NW * CPW
EP = NCHUNKP * CH
NPT = N // NS
BE = 4096

_SC_MESH = plsc.VectorSubcoreMesh(
    core_axis_name="c", subcore_axis_name="s", num_cores=NC, num_subcores=NS)
_SC_PARAMS = pltpu.CompilerParams(use_tc_tiling_on_sc=False)


def _h0_body(x_ref, w_ref, b_ref, o_ref):
    o_ref[...] = jnp.dot(x_ref[...], w_ref[...],
                         preferred_element_type=jnp.float32) + b_ref[...]


def _h0(x, fc1_W, fc1_b):
    return pl.pallas_call(
        _h0_body,
        out_shape=jax.ShapeDtypeStruct((N, W), jnp.float32),
    )(x, fc1_W, fc1_b.reshape(1, W))


def _ew_body(ea_ref, w_ref, b_ref, o_ref):
    i = pl.program_id(0)
    ew = jax.nn.relu(jnp.dot(ea_ref[...], w_ref[...],
                             preferred_element_type=jnp.float32) + b_ref[...])
    full = jnp.concatenate(
        [ew, jnp.ones((BE, 1), jnp.float32), jnp.zeros((BE, 7), jnp.float32)],
        axis=1)
    gid = i * BE + lax.broadcasted_iota(jnp.int32, (BE, 16), 0)
    o_ref[...] = jnp.where(gid < E, full, 0.0)


def _ew16(eap, k1_W, k1_b):
    return pl.pallas_call(
        _ew_body,
        grid=(EP // BE,),
        in_specs=[pl.BlockSpec((BE, 7), lambda i: (i, 0)),
                  pl.BlockSpec((7, 8), lambda i: (0, 0)),
                  pl.BlockSpec((1, 8), lambda i: (0, 0))],
        out_specs=pl.BlockSpec((BE, 16), lambda i: (i, 0)),
        out_shape=jax.ShapeDtypeStruct((EP, 16), jnp.float32),
    )(eap, k1_W, k1_b.reshape(1, 8))


def _msg_body(xj_ref, ew_ref, k_ref, o_ref):
    xj = xj_ref[...]
    ew = ew_ref[...]
    xx = jnp.concatenate([ew[:, r:r + 1] * xj for r in range(R)], axis=1)
    o_ref[...] = jnp.dot(xx, k_ref[...], preferred_element_type=jnp.float32)


def _msg(xj, ew16, kcat):
    return pl.pallas_call(
        _msg_body,
        grid=(EP // BE,),
        in_specs=[pl.BlockSpec((BE, W), lambda i: (i, 0)),
                  pl.BlockSpec((BE, 16), lambda i: (i, 0)),
                  pl.BlockSpec((R * W, W), lambda i: (0, 0))],
        out_specs=pl.BlockSpec((BE, W), lambda i: (i, 0)),
        out_shape=jax.ShapeDtypeStruct((EP, W), jnp.float32),
    )(xj, ew16, kcat)


def _recip_body(p_ref, o_ref):
    cnt = p_ref[pl.ds(0, N), :] + p_ref[pl.ds(N, N), :]
    o_ref[...] = 1.0 / jnp.maximum(cnt, 1.0)


def _recip(parts):
    return pl.pallas_call(
        _recip_body,
        out_shape=jax.ShapeDtypeStruct((N, W), jnp.float32),
    )(parts)


def _update_body(p_ref, r_ref, h_ref, w_ref, b_ref, o_ref, *, final, fw_ref=None,
                 fb_ref=None):
    sums = p_ref[pl.ds(0, N), :] + p_ref[pl.ds(N, N), :]
    u = jax.nn.relu(sums * r_ref[...]
                    + jnp.dot(h_ref[...], w_ref[...],
                              preferred_element_type=jnp.float32)
                    + b_ref[...])
    if final:
        o_ref[...] = jnp.dot(u, fw_ref[...],
                             preferred_element_type=jnp.float32) + fb_ref[...]
    else:
        o_ref[...] = u


def _upd_mid_body(p_ref, r_ref, h_ref, w_ref, b_ref, o_ref):
    _update_body(p_ref, r_ref, h_ref, w_ref, b_ref, o_ref, final=False)


def _upd_fin_body(p_ref, r_ref, h_ref, w_ref, b_ref, fw_ref, fb_ref, o_ref):
    _update_body(p_ref, r_ref, h_ref, w_ref, b_ref, o_ref, final=True,
                 fw_ref=fw_ref, fb_ref=fb_ref)


def _update(parts, recip, h, root_W, conv_b):
    return pl.pallas_call(
        _upd_mid_body,
        out_shape=jax.ShapeDtypeStruct((N, W), jnp.float32),
    )(parts, recip, h, root_W, conv_b.reshape(1, W))


def _update_final(parts, recip, h, root_W, conv_b, fc2_W, fc2_b):
    return pl.pallas_call(
        _upd_fin_body,
        out_shape=jax.ShapeDtypeStruct((N, 1), jnp.float32),
    )(parts, recip, h, root_W, conv_b.reshape(1, W), fc2_W, fc2_b.reshape(1, 1))


@functools.partial(
    pl.kernel,
    out_type=jax.ShapeDtypeStruct((NCHUNKP, CH, W), jnp.float32),
    mesh=_SC_MESH,
    scratch_types=[
        pltpu.VMEM((CPW, CH), jnp.int32),
        pltpu.VMEM((CH, W), jnp.float32),
        pltpu.SemaphoreType.DMA,
    ],
    compiler_params=_SC_PARAMS,
)
def _sc_gather(h_hbm, src_hbm, out_hbm, idx_v, rows_v, sem):
    c = lax.axis_index("c")
    s = lax.axis_index("s")
    w = c * NS + s
    base = w * CPW
    pltpu.sync_copy(src_hbm.at[pl.ds(base, CPW)], idx_v)

    def body(k, _):
        chunk = base + k
        @pl.when(chunk < NCHUNK)
        def _():
            pltpu.async_copy(h_hbm.at[idx_v.at[k]], rows_v, sem).wait()
            pltpu.sync_copy(rows_v, out_hbm.at[chunk])
        return 0

    lax.fori_loop(0, CPW, body, 0)


def _make_sc_scatter(is_cnt):
    in_types = [] if is_cnt else None

    @functools.partial(
        pl.kernel,
        out_type=jax.ShapeDtypeStruct((NC * N, W), jnp.float32),
        mesh=_SC_MESH,
        scratch_types=[
            pltpu.VMEM_SHARED((N, W), jnp.float32),
            pltpu.VMEM((CPW, CH), jnp.int32),
            pltpu.VMEM((CH, W), jnp.float32),
        ],
        compiler_params=_SC_PARAMS,
    )
    def _sc_scatter(*args):
        if is_cnt:
            dst_hbm, zeros_hbm, ones_hbm, out_hbm, acc, idx_v, rows_v = args
        else:
            msg_hbm, dst_hbm, zeros_hbm, out_hbm, acc, idx_v, rows_v = args
        c = lax.axis_index("c")
        s = lax.axis_index("s")
        w = c * NS + s
        base = w * CPW
        pltpu.sync_copy(zeros_hbm.at[pl.ds(s * NPT, NPT)],
                        acc.at[pl.ds(s * NPT, NPT)])
        plsc.subcore_barrier()
        pltpu.sync_copy(dst_hbm.at[pl.ds(base, CPW)], idx_v)
        if is_cnt:
            pltpu.sync_copy(ones_hbm, rows_v)

        def body(k, _):
            chunk = base + k
            @pl.when(chunk < NCHUNK)
            def _():
                if not is_cnt:
                    pltpu.sync_copy(msg_hbm.at[chunk], rows_v)
                pltpu.sync_copy(rows_v, acc.at[idx_v.at[k]], add=True)
            return 0

        lax.fori_loop(0, CPW, body, 0)
        plsc.subcore_barrier()
        pltpu.sync_copy(acc.at[pl.ds(s * NPT, NPT)],
                        out_hbm.at[pl.ds(c * N + s * NPT, NPT)])

    return _sc_scatter


_sc_scatter_msg = _make_sc_scatter(False)
_sc_scatter_cnt = _make_sc_scatter(True)


def kernel(x, edge_index, edge_attr, fc1_W, fc1_b, k1_W, k1_b, k2_W, k2_b,
           root_W, conv_b, fc2_W, fc2_b):
    src = edge_index[0]
    dst = edge_index[1]
    srcp = jnp.pad(src, (0, EP - E)).reshape(NCHUNKP, CH)
    dstp = jnp.pad(dst, (0, EP - E)).reshape(NCHUNKP, CH)
    eap = jnp.pad(edge_attr, ((0, EP - E), (0, 0)))
    kcat = jnp.concatenate([k2_W.reshape(8 * W, W), k2_b.reshape(W, W)], axis=0)
    zeros_nw = jnp.zeros((N, W), jnp.float32)
    ones_cw = jnp.ones((CH, W), jnp.float32)

    h = _h0(x, fc1_W, fc1_b)
    ew16 = _ew16(eap, k1_W, k1_b)
    cnt_parts = _sc_scatter_cnt(dstp, zeros_nw, ones_cw)
    recip = _recip(cnt_parts)

    for layer in range(DEPTH):
        xj = _sc_gather(h, srcp).reshape(EP, W)
        msg = _msg(xj, ew16, kcat).reshape(NCHUNKP, CH, W)
        parts = _sc_scatter_msg(msg, dstp, zeros_nw)
        if layer < DEPTH - 1:
            h = _update(parts, recip, h, root_W, conv_b)
        else:
            out = _update_final(parts, recip, h, root_W, conv_b, fc2_W, fc2_b)
    return out


# speedup vs baseline: 3.3648x; 1.5270x over previous
import functools

import jax
import jax.numpy as jnp
from jax import lax
from jax.experimental import pallas as pl
from jax.experimental.pallas import tpu as pltpu
from jax.experimental.pallas import tpu_sc as plsc

N = 10000
E = 160000
W = 32
DEPTH = 4
R = 9
NC, NS = 2, 16
NW = NC * NS
CH = 128
NCHUNK = E // CH
CPW = 40
NCHUNKP = NW * CPW
EP = NCHUNKP * CH
NPT = N // NS
BE = 4096

_SC_MESH = plsc.VectorSubcoreMesh(
    core_axis_name="c", subcore_axis_name="s", num_cores=NC, num_subcores=NS)
_SC_PARAMS = pltpu.CompilerParams(use_tc_tiling_on_sc=False)


def _h0_body(x_ref, w_ref, b_ref, o_ref):
    o_ref[...] = jnp.dot(x_ref[...], w_ref[...],
                         preferred_element_type=jnp.float32) + b_ref[...]


def _h0(x, fc1_W, fc1_b):
    return pl.pallas_call(
        _h0_body,
        out_shape=jax.ShapeDtypeStruct((N, W), jnp.float32),
    )(x, fc1_W, fc1_b.reshape(1, W))


def _ew_body(ea_ref, w_ref, b_ref, o_ref):
    i = pl.program_id(0)
    ew = jax.nn.relu(jnp.dot(ea_ref[...], w_ref[...],
                             preferred_element_type=jnp.float32) + b_ref[...])
    full = jnp.concatenate(
        [ew, jnp.ones((BE, 1), jnp.float32), jnp.zeros((BE, 7), jnp.float32)],
        axis=1)
    gid = i * BE + lax.broadcasted_iota(jnp.int32, (BE, 16), 0)
    o_ref[...] = jnp.where(gid < E, full, 0.0)


def _ew16(eap, k1_W, k1_b):
    return pl.pallas_call(
        _ew_body,
        grid=(EP // BE,),
        in_specs=[pl.BlockSpec((BE, 7), lambda i: (i, 0)),
                  pl.BlockSpec((7, 8), lambda i: (0, 0)),
                  pl.BlockSpec((1, 8), lambda i: (0, 0))],
        out_specs=pl.BlockSpec((BE, 16), lambda i: (i, 0)),
        out_shape=jax.ShapeDtypeStruct((EP, 16), jnp.float32),
    )(eap, k1_W, k1_b.reshape(1, 8))


def _msg_body(xj_ref, ew_ref, ir_ref, sr_ref, k_ref, o_ref):
    xr = jnp.dot(xj_ref[...], ir_ref[...], preferred_element_type=jnp.float32)
    er = jnp.dot(ew_ref[...], sr_ref[...], preferred_element_type=jnp.float32)
    o_ref[...] = jnp.dot(xr * er, k_ref[...],
                         preferred_element_type=jnp.float32)


def _msg(xj, ew16, irep, srep, kcat):
    return pl.pallas_call(
        _msg_body,
        grid=(EP // BE,),
        in_specs=[pl.BlockSpec((BE, W), lambda i: (i, 0)),
                  pl.BlockSpec((BE, 16), lambda i: (i, 0)),
                  pl.BlockSpec((W, R * W), lambda i: (0, 0)),
                  pl.BlockSpec((16, R * W), lambda i: (0, 0)),
                  pl.BlockSpec((R * W, W), lambda i: (0, 0))],
        out_specs=pl.BlockSpec((BE, W), lambda i: (i, 0)),
        out_shape=jax.ShapeDtypeStruct((EP, W), jnp.float32),
    )(xj, ew16, irep, srep, kcat)


def _recip_body(p_ref, o_ref):
    cnt = p_ref[pl.ds(0, N), :] + p_ref[pl.ds(N, N), :]
    o_ref[...] = 1.0 / jnp.maximum(cnt, 1.0)


def _recip(parts):
    return pl.pallas_call(
        _recip_body,
        out_shape=jax.ShapeDtypeStruct((N, W), jnp.float32),
    )(parts)


def _update_body(p_ref, r_ref, h_ref, w_ref, b_ref, o_ref, *, final, fw_ref=None,
                 fb_ref=None):
    sums = p_ref[pl.ds(0, N), :] + p_ref[pl.ds(N, N), :]
    u = jax.nn.relu(sums * r_ref[...]
                    + jnp.dot(h_ref[...], w_ref[...],
                              preferred_element_type=jnp.float32)
                    + b_ref[...])
    if final:
        o_ref[...] = jnp.dot(u, fw_ref[...],
                             preferred_element_type=jnp.float32) + fb_ref[...]
    else:
        o_ref[...] = u


def _upd_mid_body(p_ref, r_ref, h_ref, w_ref, b_ref, o_ref):
    _update_body(p_ref, r_ref, h_ref, w_ref, b_ref, o_ref, final=False)


def _upd_fin_body(p_ref, r_ref, h_ref, w_ref, b_ref, fw_ref, fb_ref, o_ref):
    _update_body(p_ref, r_ref, h_ref, w_ref, b_ref, o_ref, final=True,
                 fw_ref=fw_ref, fb_ref=fb_ref)


def _update(parts, recip, h, root_W, conv_b):
    return pl.pallas_call(
        _upd_mid_body,
        out_shape=jax.ShapeDtypeStruct((N, W), jnp.float32),
    )(parts, recip, h, root_W, conv_b.reshape(1, W))


def _update_final(parts, recip, h, root_W, conv_b, fc2_W, fc2_b):
    return pl.pallas_call(
        _upd_fin_body,
        out_shape=jax.ShapeDtypeStruct((N, 1), jnp.float32),
    )(parts, recip, h, root_W, conv_b.reshape(1, W), fc2_W, fc2_b.reshape(1, 1))


NBUF = 8


@functools.partial(
    pl.kernel,
    out_type=jax.ShapeDtypeStruct((NCHUNKP, CH, W), jnp.float32),
    mesh=_SC_MESH,
    scratch_types=[
        pltpu.VMEM((CPW, CH), jnp.int32),
        pltpu.VMEM((NBUF, CH, W), jnp.float32),
        pltpu.SemaphoreType.DMA((NBUF,)),
        pltpu.SemaphoreType.DMA((NBUF,)),
    ],
    compiler_params=_SC_PARAMS,
)
def _sc_gather(h_hbm, src_hbm, out_hbm, idx_v, rows_v, gsem, wsem):
    c = lax.axis_index("c")
    s = lax.axis_index("s")
    w = c * NS + s
    base = w * CPW
    pltpu.sync_copy(src_hbm.at[pl.ds(base, CPW)], idx_v)

    for b in range(NBUF):
        pltpu.async_copy(h_hbm.at[idx_v.at[b]], rows_v.at[b], gsem.at[b])

    def body(k, _):
        slot = lax.rem(k, NBUF)
        pltpu.make_async_copy(h_hbm.at[idx_v.at[k]], rows_v.at[slot],
                              gsem.at[slot]).wait()
        pltpu.async_copy(rows_v.at[slot], out_hbm.at[base + k], wsem.at[slot])
        nk = k + NBUF

        @pl.when(nk < CPW)
        def _():
            pltpu.make_async_copy(rows_v.at[slot], out_hbm.at[base + k],
                                  wsem.at[slot]).wait()
            pltpu.async_copy(h_hbm.at[idx_v.at[nk]], rows_v.at[slot],
                             gsem.at[slot])
        return 0

    lax.fori_loop(0, CPW, body, 0)

    def drain(k, _):
        slot = lax.rem(k, NBUF)
        pltpu.make_async_copy(rows_v.at[slot], out_hbm.at[base + k],
                              wsem.at[slot]).wait()
        return 0

    lax.fori_loop(CPW - NBUF, CPW, drain, 0)


def _make_sc_scatter(is_cnt):
    in_types = [] if is_cnt else None

    @functools.partial(
        pl.kernel,
        out_type=jax.ShapeDtypeStruct((NC * N, W), jnp.float32),
        mesh=_SC_MESH,
        scratch_types=[
            pltpu.VMEM_SHARED((N, W), jnp.float32),
            pltpu.VMEM((CPW, CH), jnp.int32),
            pltpu.VMEM((NBUF, CH, W), jnp.float32),
            pltpu.SemaphoreType.DMA((NBUF,)),
            pltpu.SemaphoreType.DMA((NBUF,)),
        ],
        compiler_params=_SC_PARAMS,
    )
    def _sc_scatter(*args):
        if is_cnt:
            dst_hbm, zeros_hbm, ones_hbm, out_hbm, acc, idx_v, rows_v, lsem, ssem = args
        else:
            msg_hbm, dst_hbm, zeros_hbm, out_hbm, acc, idx_v, rows_v, lsem, ssem = args
        c = lax.axis_index("c")
        s = lax.axis_index("s")
        w = c * NS + s
        base = w * CPW
        pltpu.sync_copy(zeros_hbm.at[pl.ds(s * NPT, NPT)],
                        acc.at[pl.ds(s * NPT, NPT)])
        plsc.subcore_barrier()
        pltpu.sync_copy(dst_hbm.at[pl.ds(base, CPW)], idx_v)

        if is_cnt:
            pltpu.sync_copy(ones_hbm, rows_v.at[0])

            def body(k, _):
                @pl.when(base + k < NCHUNK)
                def _():
                    pltpu.sync_copy(rows_v.at[0], acc.at[idx_v.at[k]], add=True)
                return 0

            lax.fori_loop(0, CPW, body, 0)
        else:
            for b in range(NBUF):
                pltpu.async_copy(msg_hbm.at[base + b], rows_v.at[b], lsem.at[b])

            def body(k, _):
                slot = lax.rem(k, NBUF)
                pltpu.make_async_copy(msg_hbm.at[base + k], rows_v.at[slot],
                                      lsem.at[slot]).wait()
                pltpu.async_copy(rows_v.at[slot], acc.at[idx_v.at[k]],
                                 ssem.at[slot], add=True)
                nk = k + NBUF

                @pl.when(nk < CPW)
                def _():
                    pltpu.make_async_copy(rows_v.at[slot], acc.at[idx_v.at[k]],
                                          ssem.at[slot]).wait()
                    pltpu.async_copy(msg_hbm.at[base + nk], rows_v.at[slot],
                                     lsem.at[slot])
                return 0

            lax.fori_loop(0, CPW, body, 0)

            def drain(k, _):
                slot = lax.rem(k, NBUF)
                pltpu.make_async_copy(rows_v.at[slot], acc.at[idx_v.at[k]],
                                      ssem.at[slot]).wait()
                return 0

            lax.fori_loop(CPW - NBUF, CPW, drain, 0)

        plsc.subcore_barrier()
        pltpu.sync_copy(acc.at[pl.ds(s * NPT, NPT)],
                        out_hbm.at[pl.ds(c * N + s * NPT, NPT)])

    return _sc_scatter


_sc_scatter_msg = _make_sc_scatter(False)
_sc_scatter_cnt = _make_sc_scatter(True)


def kernel(x, edge_index, edge_attr, fc1_W, fc1_b, k1_W, k1_b, k2_W, k2_b,
           root_W, conv_b, fc2_W, fc2_b):
    src = edge_index[0]
    dst = edge_index[1]
    srcp = jnp.pad(src, (0, EP - E)).reshape(NCHUNKP, CH)
    dstp = jnp.pad(dst, (0, EP - E)).reshape(NCHUNKP, CH)
    eap = jnp.pad(edge_attr, ((0, EP - E), (0, 0)))
    kcat = jnp.concatenate([k2_W.reshape(8 * W, W), k2_b.reshape(W, W)], axis=0)
    irep = jnp.tile(jnp.eye(W, dtype=jnp.float32), (1, R))
    srep = jnp.repeat(jnp.eye(16, dtype=jnp.float32), W, axis=1)[:, :R * W]
    zeros_nw = jnp.zeros((N, W), jnp.float32)
    ones_cw = jnp.ones((CH, W), jnp.float32)

    h = _h0(x, fc1_W, fc1_b)
    ew16 = _ew16(eap, k1_W, k1_b)
    cnt_parts = _sc_scatter_cnt(dstp, zeros_nw, ones_cw)
    recip = _recip(cnt_parts)

    for layer in range(DEPTH):
        xj = _sc_gather(h, srcp).reshape(EP, W)
        msg = _msg(xj, ew16, irep, srep, kcat).reshape(NCHUNKP, CH, W)
        parts = _sc_scatter_msg(msg, dstp, zeros_nw)
        if layer < DEPTH - 1:
            h = _update(parts, recip, h, root_W, conv_b)
        else:
            out = _update_final(parts, recip, h, root_W, conv_b, fc2_W, fc2_b)
    return out


# speedup vs baseline: 3.9987x; 1.1884x over previous
import functools

import jax
import jax.numpy as jnp
from jax import lax
from jax.experimental import pallas as pl
from jax.experimental.pallas import tpu as pltpu
from jax.experimental.pallas import tpu_sc as plsc

N = 10000
E = 160000
W = 32
DEPTH = 4
R = 9
NC, NS = 2, 16
NW = NC * NS
CH = 125
NCHUNK = E // CH
CPW = NCHUNK // NW
SUP = 8
NSUP = CPW // SUP
SUPE = SUP * CH
NPT = N // NS
BE = 4000

_SC_MESH = plsc.VectorSubcoreMesh(
    core_axis_name="c", subcore_axis_name="s", num_cores=NC, num_subcores=NS)
_SC_PARAMS = pltpu.CompilerParams(use_tc_tiling_on_sc=False)


def _h0_body(x_ref, w_ref, b_ref, o_ref):
    o_ref[...] = jnp.dot(x_ref[...], w_ref[...],
                         preferred_element_type=jnp.float32) + b_ref[...]


def _h0(x, fc1_W, fc1_b):
    return pl.pallas_call(
        _h0_body,
        out_shape=jax.ShapeDtypeStruct((N, W), jnp.float32),
    )(x, fc1_W, fc1_b.reshape(1, W))


def _ew_body(ea_ref, w_ref, b_ref, o_ref):
    ew = jax.nn.relu(jnp.dot(ea_ref[...], w_ref[...],
                             preferred_element_type=jnp.float32) + b_ref[...])
    o_ref[...] = jnp.concatenate(
        [ew, jnp.ones((BE, 1), jnp.float32), jnp.zeros((BE, 7), jnp.float32)],
        axis=1)


def _ew16(edge_attr, k1_W, k1_b):
    return pl.pallas_call(
        _ew_body,
        grid=(E // BE,),
        in_specs=[pl.BlockSpec((BE, 7), lambda i: (i, 0)),
                  pl.BlockSpec((7, 8), lambda i: (0, 0)),
                  pl.BlockSpec((1, 8), lambda i: (0, 0))],
        out_specs=pl.BlockSpec((BE, 16), lambda i: (i, 0)),
        out_shape=jax.ShapeDtypeStruct((E, 16), jnp.float32),
    )(edge_attr, k1_W, k1_b.reshape(1, 8))


def _msg_body(xj_ref, ew_ref, ir_ref, sr_ref, k_ref, o_ref):
    xr = jnp.dot(xj_ref[...], ir_ref[...], preferred_element_type=jnp.float32)
    er = jnp.dot(ew_ref[...], sr_ref[...], preferred_element_type=jnp.float32)
    o_ref[...] = jnp.dot(xr * er, k_ref[...],
                         preferred_element_type=jnp.float32)


def _msg(xj, ew16, irep, srep, kcat):
    return pl.pallas_call(
        _msg_body,
        grid=(E // BE,),
        in_specs=[pl.BlockSpec((BE, W), lambda i: (i, 0)),
                  pl.BlockSpec((BE, 16), lambda i: (i, 0)),
                  pl.BlockSpec((W, R * W), lambda i: (0, 0)),
                  pl.BlockSpec((16, R * W), lambda i: (0, 0)),
                  pl.BlockSpec((R * W, W), lambda i: (0, 0))],
        out_specs=pl.BlockSpec((BE, W), lambda i: (i, 0)),
        out_shape=jax.ShapeDtypeStruct((E, W), jnp.float32),
    )(xj, ew16, irep, srep, kcat)


def _recip_body(p_ref, o_ref):
    cnt = p_ref[pl.ds(0, N), :] + p_ref[pl.ds(N, N), :]
    o_ref[...] = 1.0 / jnp.maximum(cnt, 1.0)


def _recip(parts):
    return pl.pallas_call(
        _recip_body,
        out_shape=jax.ShapeDtypeStruct((N, W), jnp.float32),
    )(parts)


def _update_body(p_ref, r_ref, h_ref, w_ref, b_ref, o_ref, *, final, fw_ref=None,
                 fb_ref=None):
    sums = p_ref[pl.ds(0, N), :] + p_ref[pl.ds(N, N), :]
    u = jax.nn.relu(sums * r_ref[...]
                    + jnp.dot(h_ref[...], w_ref[...],
                              preferred_element_type=jnp.float32)
                    + b_ref[...])
    if final:
        o_ref[...] = jnp.dot(u, fw_ref[...],
                             preferred_element_type=jnp.float32) + fb_ref[...]
    else:
        o_ref[...] = u


def _upd_mid_body(p_ref, r_ref, h_ref, w_ref, b_ref, o_ref):
    _update_body(p_ref, r_ref, h_ref, w_ref, b_ref, o_ref, final=False)


def _upd_fin_body(p_ref, r_ref, h_ref, w_ref, b_ref, fw_ref, fb_ref, o_ref):
    _update_body(p_ref, r_ref, h_ref, w_ref, b_ref, o_ref, final=True,
                 fw_ref=fw_ref, fb_ref=fb_ref)


def _update(parts, recip, h, root_W, conv_b):
    return pl.pallas_call(
        _upd_mid_body,
        out_shape=jax.ShapeDtypeStruct((N, W), jnp.float32),
    )(parts, recip, h, root_W, conv_b.reshape(1, W))


def _update_final(parts, recip, h, root_W, conv_b, fc2_W, fc2_b):
    return pl.pallas_call(
        _upd_fin_body,
        out_shape=jax.ShapeDtypeStruct((N, 1), jnp.float32),
    )(parts, recip, h, root_W, conv_b.reshape(1, W), fc2_W, fc2_b.reshape(1, 1))


@functools.partial(
    pl.kernel,
    out_type=jax.ShapeDtypeStruct((E, W), jnp.float32),
    mesh=_SC_MESH,
    scratch_types=[
        pltpu.VMEM((CPW, CH), jnp.int32),
        pltpu.VMEM((2, SUPE, W), jnp.float32),
        pltpu.SemaphoreType.DMA((2,)),
        pltpu.SemaphoreType.DMA((2,)),
    ],
    compiler_params=_SC_PARAMS,
)
def _sc_gather(h_hbm, src_hbm, out_hbm, idx_v, big_v, gsem, wsem):
    c = lax.axis_index("c")
    s = lax.axis_index("s")
    w = c * NS + s
    base = w * CPW
    pltpu.sync_copy(src_hbm.at[pl.ds(base, CPW)], idx_v)

    def fire_super(sidx, buf):
        for j in range(SUP):
            pltpu.async_copy(h_hbm.at[idx_v.at[sidx * SUP + j]],
                             big_v.at[buf, pl.ds(j * CH, CH)], gsem.at[buf])

    fire_super(0, 0)

    def body(sidx, _):
        buf = lax.rem(sidx, 2)
        obuf = 1 - buf

        @pl.when(sidx + 1 < NSUP)
        def _():
            @pl.when(sidx >= 1)
            def _():
                pltpu.make_async_copy(big_v.at[obuf],
                                      out_hbm.at[pl.ds(0, SUPE)],
                                      wsem.at[obuf]).wait()
            fire_super(sidx + 1, obuf)

        for j in range(SUP):
            pltpu.make_async_copy(h_hbm.at[idx_v.at[0]],
                                  big_v.at[buf, pl.ds(0, CH)],
                                  gsem.at[buf]).wait()
        pltpu.async_copy(big_v.at[buf],
                         out_hbm.at[pl.ds((base + sidx * SUP) * CH, SUPE)],
                         wsem.at[buf])
        return 0

    lax.fori_loop(0, NSUP, body, 0)
    for t in (NSUP - 2, NSUP - 1):
        pltpu.make_async_copy(big_v.at[t % 2], out_hbm.at[pl.ds(0, SUPE)],
                              wsem.at[t % 2]).wait()


def _make_sc_scatter(is_cnt):
    @functools.partial(
        pl.kernel,
        out_type=jax.ShapeDtypeStruct((NC * N, W), jnp.float32),
        mesh=_SC_MESH,
        scratch_types=[
            pltpu.VMEM_SHARED((N, W), jnp.float32),
            pltpu.VMEM((CPW, CH), jnp.int32),
            pltpu.VMEM((2, SUPE, W), jnp.float32),
            pltpu.SemaphoreType.DMA((2,)),
            pltpu.SemaphoreType.DMA((2,)),
        ],
        compiler_params=_SC_PARAMS,
    )
    def _sc_scatter(*args):
        if is_cnt:
            dst_hbm, zeros_hbm, ones_hbm, out_hbm, acc, idx_v, big_v, lsem, asem = args
        else:
            msg_hbm, dst_hbm, zeros_hbm, out_hbm, acc, idx_v, big_v, lsem, asem = args
        c = lax.axis_index("c")
        s = lax.axis_index("s")
        w = c * NS + s
        base = w * CPW
        pltpu.sync_copy(zeros_hbm.at[pl.ds(s * NPT, NPT)],
                        acc.at[pl.ds(s * NPT, NPT)])
        plsc.subcore_barrier()
        pltpu.sync_copy(dst_hbm.at[pl.ds(base, CPW)], idx_v)

        def fire_adds(sidx, buf):
            for j in range(SUP):
                pltpu.async_copy(big_v.at[buf, pl.ds(j * CH, CH)],
                                 acc.at[idx_v.at[sidx * SUP + j]],
                                 asem.at[buf], add=True)

        def drain_adds(buf):
            for j in range(SUP):
                pltpu.make_async_copy(big_v.at[buf, pl.ds(0, CH)],
                                      acc.at[idx_v.at[0]], asem.at[buf]).wait()

        if is_cnt:
            pltpu.sync_copy(ones_hbm, big_v.at[0, pl.ds(0, CH)])

            def body(sidx, _):
                for j in range(SUP):
                    pltpu.async_copy(big_v.at[0, pl.ds(0, CH)],
                                     acc.at[idx_v.at[sidx * SUP + j]],
                                     asem.at[0], add=True)
                drain_adds(0)
                return 0

            lax.fori_loop(0, NSUP, body, 0)
        else:
            pltpu.async_copy(msg_hbm.at[pl.ds(base * CH, SUPE)], big_v.at[0],
                             lsem.at[0])

            def body(sidx, _):
                buf = lax.rem(sidx, 2)
                obuf = 1 - buf
                pltpu.make_async_copy(msg_hbm.at[pl.ds(0, SUPE)],
                                      big_v.at[buf], lsem.at[buf]).wait()

                @pl.when(sidx + 1 < NSUP)
                def _():
                    @pl.when(sidx >= 1)
                    def _():
                        drain_adds(obuf)
                    pltpu.async_copy(
                        msg_hbm.at[pl.ds((base + (sidx + 1) * SUP) * CH, SUPE)],
                        big_v.at[obuf], lsem.at[obuf])

                fire_adds(sidx, buf)
                return 0

            lax.fori_loop(0, NSUP, body, 0)
            for t in (NSUP - 2, NSUP - 1):
                drain_adds(t % 2)

        plsc.subcore_barrier()
        pltpu.sync_copy(acc.at[pl.ds(s * NPT, NPT)],
                        out_hbm.at[pl.ds(c * N + s * NPT, NPT)])

    return _sc_scatter


_sc_scatter_msg = _make_sc_scatter(False)
_sc_scatter_cnt = _make_sc_scatter(True)


def kernel(x, edge_index, edge_attr, fc1_W, fc1_b, k1_W, k1_b, k2_W, k2_b,
           root_W, conv_b, fc2_W, fc2_b):
    srcp = edge_index[0].reshape(NCHUNK, CH)
    dstp = edge_index[1].reshape(NCHUNK, CH)
    kcat = jnp.concatenate([k2_W.reshape(8 * W, W), k2_b.reshape(W, W)], axis=0)
    irep = jnp.tile(jnp.eye(W, dtype=jnp.float32), (1, R))
    srep = jnp.repeat(jnp.eye(16, dtype=jnp.float32), W, axis=1)[:, :R * W]
    zeros_nw = jnp.zeros((N, W), jnp.float32)
    ones_cw = jnp.ones((CH, W), jnp.float32)

    h = _h0(x, fc1_W, fc1_b)
    ew16 = _ew16(edge_attr, k1_W, k1_b)
    cnt_parts = _sc_scatter_cnt(dstp, zeros_nw, ones_cw)
    recip = _recip(cnt_parts)

    for layer in range(DEPTH):
        xj = _sc_gather(h, srcp)
        msg = _msg(xj, ew16, irep, srep, kcat)
        parts = _sc_scatter_msg(msg, dstp, zeros_nw)
        if layer < DEPTH - 1:
            h = _update(parts, recip, h, root_W, conv_b)
        else:
            out = _update_final(parts, recip, h, root_W, conv_b, fc2_W, fc2_b)
    return out


# speedup vs baseline: 5.7640x; 1.4415x over previous
import functools

import jax
import jax.numpy as jnp
from jax import lax
from jax.experimental import pallas as pl
from jax.experimental.pallas import tpu as pltpu
from jax.experimental.pallas import tpu_sc as plsc

N = 10000
E = 160000
W = 32
DEPTH = 4
R = 9
NC, NS = 2, 16
NW = NC * NS
CH = 125
NCHUNK = E // CH
CPW = NCHUNK // NW
SUP = 8
NSUP = CPW // SUP
SUPE = SUP * CH
NPT = N // NS
BE = 4000

_SC_MESH = plsc.VectorSubcoreMesh(
    core_axis_name="c", subcore_axis_name="s", num_cores=NC, num_subcores=NS)
_SC_PARAMS = pltpu.CompilerParams(use_tc_tiling_on_sc=False)


def _h0_body(x_ref, w_ref, b_ref, o_ref):
    o_ref[...] = jnp.dot(x_ref[...], w_ref[...],
                         preferred_element_type=jnp.float32) + b_ref[...]


def _h0(x, fc1_W, fc1_b):
    return pl.pallas_call(
        _h0_body,
        out_shape=jax.ShapeDtypeStruct((N, W), jnp.float32),
    )(x, fc1_W, fc1_b.reshape(1, W))


def _ew_body(ea_ref, w_ref, b_ref, m_ref, o_ref):
    u = jnp.dot(ea_ref[...], w_ref[...],
                preferred_element_type=jnp.float32) + b_ref[...]
    o_ref[...] = jax.nn.relu(u) + m_ref[...]


def _ew16(ea4, k1_W4, k1_b4, ones_mask):
    b4 = BE // 4
    return pl.pallas_call(
        _ew_body,
        grid=(E // BE,),
        in_specs=[pl.BlockSpec((b4, 28), lambda i: (i, 0)),
                  pl.BlockSpec((28, 64), lambda i: (0, 0)),
                  pl.BlockSpec((1, 64), lambda i: (0, 0)),
                  pl.BlockSpec((1, 64), lambda i: (0, 0))],
        out_specs=pl.BlockSpec((b4, 64), lambda i: (i, 0)),
        out_shape=jax.ShapeDtypeStruct((E // 4, 64), jnp.float32),
    )(ea4, k1_W4, k1_b4, ones_mask)


def _msg_body(xj_ref, ew_ref, ir_ref, sr_ref, k_ref, o_ref):
    bf = jnp.bfloat16
    xr = jnp.dot(xj_ref[...].astype(bf), ir_ref[...],
                 preferred_element_type=jnp.float32)
    er = jnp.dot(ew_ref[...].astype(bf), sr_ref[...],
                 preferred_element_type=jnp.float32)
    o_ref[...] = jnp.dot((xr * er).astype(bf), k_ref[...],
                         preferred_element_type=jnp.float32)


def _msg(xj4, ew4, irep4, srep4, kcat4):
    b4 = BE // 4
    return pl.pallas_call(
        _msg_body,
        grid=(E // BE,),
        in_specs=[pl.BlockSpec((b4, 128), lambda i: (i, 0)),
                  pl.BlockSpec((b4, 64), lambda i: (i, 0)),
                  pl.BlockSpec((128, 4 * R * W), lambda i: (0, 0)),
                  pl.BlockSpec((64, 4 * R * W), lambda i: (0, 0)),
                  pl.BlockSpec((4 * R * W, 128), lambda i: (0, 0))],
        out_specs=pl.BlockSpec((b4, 128), lambda i: (i, 0)),
        out_shape=jax.ShapeDtypeStruct((E // 4, 128), jnp.float32),
    )(xj4, ew4, irep4, srep4, kcat4)


def _recip_body(p_ref, o_ref):
    cnt = p_ref[pl.ds(0, N), :] + p_ref[pl.ds(N, N), :]
    o_ref[...] = 1.0 / jnp.maximum(cnt, 1.0)


def _recip(parts):
    return pl.pallas_call(
        _recip_body,
        out_shape=jax.ShapeDtypeStruct((N, W), jnp.float32),
    )(parts)


def _update_body(p_ref, r_ref, h_ref, w_ref, b_ref, o_ref, *, final, fw_ref=None,
                 fb_ref=None):
    sums = p_ref[pl.ds(0, N), :] + p_ref[pl.ds(N, N), :]
    u = jax.nn.relu(sums * r_ref[...]
                    + jnp.dot(h_ref[...], w_ref[...],
                              preferred_element_type=jnp.float32)
                    + b_ref[...])
    if final:
        o_ref[...] = jnp.dot(u, fw_ref[...],
                             preferred_element_type=jnp.float32) + fb_ref[...]
    else:
        o_ref[...] = u


def _upd_mid_body(p_ref, r_ref, h_ref, w_ref, b_ref, o_ref):
    _update_body(p_ref, r_ref, h_ref, w_ref, b_ref, o_ref, final=False)


def _upd_fin_body(p_ref, r_ref, h_ref, w_ref, b_ref, fw_ref, fb_ref, o_ref):
    _update_body(p_ref, r_ref, h_ref, w_ref, b_ref, o_ref, final=True,
                 fw_ref=fw_ref, fb_ref=fb_ref)


def _update(parts, recip, h, root_W, conv_b):
    return pl.pallas_call(
        _upd_mid_body,
        out_shape=jax.ShapeDtypeStruct((N, W), jnp.float32),
    )(parts, recip, h, root_W, conv_b.reshape(1, W))


def _update_final(parts, recip, h, root_W, conv_b, fc2_W, fc2_b):
    return pl.pallas_call(
        _upd_fin_body,
        out_shape=jax.ShapeDtypeStruct((N, 1), jnp.float32),
    )(parts, recip, h, root_W, conv_b.reshape(1, W), fc2_W, fc2_b.reshape(1, 1))


@functools.partial(
    pl.kernel,
    out_type=jax.ShapeDtypeStruct((E, W), jnp.float32),
    mesh=_SC_MESH,
    scratch_types=[
        pltpu.VMEM((CPW, CH), jnp.int32),
        pltpu.VMEM((2, SUPE, W), jnp.float32),
        pltpu.SemaphoreType.DMA((2,)),
        pltpu.SemaphoreType.DMA((2,)),
    ],
    compiler_params=_SC_PARAMS,
)
def _sc_gather(h_hbm, src_hbm, out_hbm, idx_v, big_v, gsem, wsem):
    c = lax.axis_index("c")
    s = lax.axis_index("s")
    w = c * NS + s
    base = w * CPW
    pltpu.sync_copy(src_hbm.at[pl.ds(base, CPW)], idx_v)

    def fire_super(sidx, buf):
        for j in range(SUP):
            pltpu.async_copy(h_hbm.at[idx_v.at[sidx * SUP + j]],
                             big_v.at[buf, pl.ds(j * CH, CH)], gsem.at[buf])

    fire_super(0, 0)

    def body(sidx, _):
        buf = lax.rem(sidx, 2)
        obuf = 1 - buf

        @pl.when(sidx + 1 < NSUP)
        def _():
            @pl.when(sidx >= 1)
            def _():
                pltpu.make_async_copy(big_v.at[obuf],
                                      out_hbm.at[pl.ds(0, SUPE)],
                                      wsem.at[obuf]).wait()
            fire_super(sidx + 1, obuf)

        for j in range(SUP):
            pltpu.make_async_copy(h_hbm.at[idx_v.at[0]],
                                  big_v.at[buf, pl.ds(0, CH)],
                                  gsem.at[buf]).wait()
        pltpu.async_copy(big_v.at[buf],
                         out_hbm.at[pl.ds((base + sidx * SUP) * CH, SUPE)],
                         wsem.at[buf])
        return 0

    lax.fori_loop(0, NSUP, body, 0)
    for t in (NSUP - 2, NSUP - 1):
        pltpu.make_async_copy(big_v.at[t % 2], out_hbm.at[pl.ds(0, SUPE)],
                              wsem.at[t % 2]).wait()


def _make_sc_scatter(is_cnt):
    @functools.partial(
        pl.kernel,
        out_type=jax.ShapeDtypeStruct((NC * N, W), jnp.float32),
        mesh=_SC_MESH,
        scratch_types=[
            pltpu.VMEM_SHARED((N, W), jnp.float32),
            pltpu.VMEM((CPW, CH), jnp.int32),
            pltpu.VMEM((2, SUPE, W), jnp.float32),
            pltpu.SemaphoreType.DMA((2,)),
            pltpu.SemaphoreType.DMA((2,)),
        ],
        compiler_params=_SC_PARAMS,
    )
    def _sc_scatter(*args):
        if is_cnt:
            dst_hbm, zeros_hbm, ones_hbm, out_hbm, acc, idx_v, big_v, lsem, asem = args
        else:
            msg_hbm, dst_hbm, zeros_hbm, out_hbm, acc, idx_v, big_v, lsem, asem = args
        c = lax.axis_index("c")
        s = lax.axis_index("s")
        w = c * NS + s
        base = w * CPW
        pltpu.sync_copy(zeros_hbm.at[pl.ds(s * NPT, NPT)],
                        acc.at[pl.ds(s * NPT, NPT)])
        plsc.subcore_barrier()
        pltpu.sync_copy(dst_hbm.at[pl.ds(base, CPW)], idx_v)

        def fire_adds(sidx, buf):
            for j in range(SUP):
                pltpu.async_copy(big_v.at[buf, pl.ds(j * CH, CH)],
                                 acc.at[idx_v.at[sidx * SUP + j]],
                                 asem.at[buf], add=True)

        def drain_adds(buf):
            for j in range(SUP):
                pltpu.make_async_copy(big_v.at[buf, pl.ds(0, CH)],
                                      acc.at[idx_v.at[0]], asem.at[buf]).wait()

        if is_cnt:
            pltpu.sync_copy(ones_hbm, big_v.at[0, pl.ds(0, CH)])

            def body(sidx, _):
                for j in range(SUP):
                    pltpu.async_copy(big_v.at[0, pl.ds(0, CH)],
                                     acc.at[idx_v.at[sidx * SUP + j]],
                                     asem.at[0], add=True)
                drain_adds(0)
                return 0

            lax.fori_loop(0, NSUP, body, 0)
        else:
            pltpu.async_copy(msg_hbm.at[pl.ds(base * CH, SUPE)], big_v.at[0],
                             lsem.at[0])

            def body(sidx, _):
                buf = lax.rem(sidx, 2)
                obuf = 1 - buf
                pltpu.make_async_copy(msg_hbm.at[pl.ds(0, SUPE)],
                                      big_v.at[buf], lsem.at[buf]).wait()

                @pl.when(sidx + 1 < NSUP)
                def _():
                    @pl.when(sidx >= 1)
                    def _():
                        drain_adds(obuf)
                    pltpu.async_copy(
                        msg_hbm.at[pl.ds((base + (sidx + 1) * SUP) * CH, SUPE)],
                        big_v.at[obuf], lsem.at[obuf])

                fire_adds(sidx, buf)
                return 0

            lax.fori_loop(0, NSUP, body, 0)
            for t in (NSUP - 2, NSUP - 1):
                drain_adds(t % 2)

        plsc.subcore_barrier()
        pltpu.sync_copy(acc.at[pl.ds(s * NPT, NPT)],
                        out_hbm.at[pl.ds(c * N + s * NPT, NPT)])

    return _sc_scatter


_sc_scatter_msg = _make_sc_scatter(False)
_sc_scatter_cnt = _make_sc_scatter(True)


def kernel(x, edge_index, edge_attr, fc1_W, fc1_b, k1_W, k1_b, k2_W, k2_b,
           root_W, conv_b, fc2_W, fc2_b):
    srcp = edge_index[0].reshape(NCHUNK, CH)
    dstp = edge_index[1].reshape(NCHUNK, CH)
    kcat = jnp.concatenate([k2_W.reshape(8 * W, W), k2_b.reshape(W, W)], axis=0)
    irep = jnp.tile(jnp.eye(W, dtype=jnp.float32), (1, R))
    srep = jnp.repeat(jnp.eye(16, dtype=jnp.float32), W, axis=1)[:, :R * W]
    eye4 = jnp.eye(4, dtype=jnp.float32)
    irep4 = jnp.kron(eye4, irep).astype(jnp.bfloat16)
    srep4 = jnp.kron(eye4, srep).astype(jnp.bfloat16)
    kcat4 = jnp.kron(eye4, kcat).astype(jnp.bfloat16)
    k1x = jnp.pad(k1_W, ((0, 0), (0, 8)))
    k1_W4 = jnp.kron(eye4, k1x)
    k1_b4 = jnp.tile(jnp.pad(k1_b, (0, 8)), 4).reshape(1, 64)
    lane64 = jnp.arange(64) % 16
    ones_mask = (lane64 == 8).astype(jnp.float32).reshape(1, 64)
    zeros_nw = jnp.zeros((N, W), jnp.float32)
    ones_cw = jnp.ones((CH, W), jnp.float32)

    h = _h0(x, fc1_W, fc1_b)
    ew4 = _ew16(edge_attr.reshape(E // 4, 28), k1_W4, k1_b4, ones_mask)
    cnt_parts = _sc_scatter_cnt(dstp, zeros_nw, ones_cw)
    recip = _recip(cnt_parts)

    for layer in range(DEPTH):
        xj = _sc_gather(h, srcp)
        msg4 = _msg(xj.reshape(E // 4, 128), ew4, irep4, srep4, kcat4)
        parts = _sc_scatter_msg(msg4.reshape(E, W), dstp, zeros_nw)
        if layer < DEPTH - 1:
            h = _update(parts, recip, h, root_W, conv_b)
        else:
            out = _update_final(parts, recip, h, root_W, conv_b, fc2_W, fc2_b)
    return out


# speedup vs baseline: 6.0546x; 1.0504x over previous
import functools

import jax
import jax.numpy as jnp
from jax import lax
from jax.experimental import pallas as pl
from jax.experimental.pallas import tpu as pltpu
from jax.experimental.pallas import tpu_sc as plsc

N = 10000
E = 160000
W = 32
DEPTH = 4
R = 9
NC, NS = 2, 16
NW = NC * NS
CH = 125
NCHUNK = E // CH
CPW = NCHUNK // NW
SUP = 8
NSUP = CPW // SUP
SUPE = SUP * CH
NPT = N // NS
BE = 8000

_SC_MESH = plsc.VectorSubcoreMesh(
    core_axis_name="c", subcore_axis_name="s", num_cores=NC, num_subcores=NS)
_SC_PARAMS = pltpu.CompilerParams(use_tc_tiling_on_sc=False)


def _h0_body(x_ref, w_ref, b_ref, o_ref):
    o_ref[...] = jnp.dot(x_ref[...], w_ref[...],
                         preferred_element_type=jnp.float32) + b_ref[...]


def _h0(x, fc1_W, fc1_b):
    return pl.pallas_call(
        _h0_body,
        out_shape=jax.ShapeDtypeStruct((N, W), jnp.float32),
    )(x, fc1_W, fc1_b.reshape(1, W))


def _ew_body(ea_ref, w_ref, b_ref, m_ref, o_ref):
    u = jnp.dot(ea_ref[...], w_ref[...],
                preferred_element_type=jnp.float32) + b_ref[...]
    o_ref[...] = jax.nn.relu(u) + m_ref[...]


def _ew16(ea4, k1_W4, k1_b4, ones_mask):
    b4 = BE // 4
    return pl.pallas_call(
        _ew_body,
        grid=(E // BE,),
        in_specs=[pl.BlockSpec((b4, 28), lambda i: (i, 0)),
                  pl.BlockSpec((28, 64), lambda i: (0, 0)),
                  pl.BlockSpec((1, 64), lambda i: (0, 0)),
                  pl.BlockSpec((1, 64), lambda i: (0, 0))],
        out_specs=pl.BlockSpec((b4, 64), lambda i: (i, 0)),
        out_shape=jax.ShapeDtypeStruct((E // 4, 64), jnp.float32),
    )(ea4, k1_W4, k1_b4, ones_mask)


def _msg_body(xj_ref, ew_ref, s_ref, k_ref, o_ref):
    bf = jnp.bfloat16
    xj = xj_ref[...].astype(bf)
    ew = ew_ref[...].astype(bf)
    acc = jnp.zeros((BE // 4, 128), jnp.float32)
    for r in range(R):
        er = jnp.dot(ew, s_ref[r],
                     preferred_element_type=jnp.float32).astype(bf)
        acc = acc + jnp.dot(xj * er, k_ref[r],
                            preferred_element_type=jnp.float32)
    o_ref[...] = acc


def _msg(xj4, ew4, sreps, kcats):
    b4 = BE // 4
    return pl.pallas_call(
        _msg_body,
        grid=(E // BE,),
        in_specs=[pl.BlockSpec((b4, 128), lambda i: (i, 0)),
                  pl.BlockSpec((b4, 64), lambda i: (i, 0)),
                  pl.BlockSpec((R, 64, 128), lambda i: (0, 0, 0)),
                  pl.BlockSpec((R, 128, 128), lambda i: (0, 0, 0))],
        out_specs=pl.BlockSpec((b4, 128), lambda i: (i, 0)),
        out_shape=jax.ShapeDtypeStruct((E // 4, 128), jnp.float32),
    )(xj4, ew4, sreps, kcats)


def _recip_body(p_ref, o_ref):
    cnt = p_ref[pl.ds(0, N), :] + p_ref[pl.ds(N, N), :]
    o_ref[...] = 1.0 / jnp.maximum(cnt, 1.0)


def _recip(parts):
    return pl.pallas_call(
        _recip_body,
        out_shape=jax.ShapeDtypeStruct((N, W), jnp.float32),
    )(parts)


def _update_body(p_ref, r_ref, h_ref, w_ref, b_ref, o_ref, *, final, fw_ref=None,
                 fb_ref=None):
    sums = p_ref[pl.ds(0, N), :] + p_ref[pl.ds(N, N), :]
    u = jax.nn.relu(sums * r_ref[...]
                    + jnp.dot(h_ref[...], w_ref[...],
                              preferred_element_type=jnp.float32)
                    + b_ref[...])
    if final:
        o_ref[...] = jnp.dot(u, fw_ref[...],
                             preferred_element_type=jnp.float32) + fb_ref[...]
    else:
        o_ref[...] = u


def _upd_mid_body(p_ref, r_ref, h_ref, w_ref, b_ref, o_ref):
    _update_body(p_ref, r_ref, h_ref, w_ref, b_ref, o_ref, final=False)


def _upd_fin_body(p_ref, r_ref, h_ref, w_ref, b_ref, fw_ref, fb_ref, o_ref):
    _update_body(p_ref, r_ref, h_ref, w_ref, b_ref, o_ref, final=True,
                 fw_ref=fw_ref, fb_ref=fb_ref)


def _update(parts, recip, h, root_W, conv_b):
    return pl.pallas_call(
        _upd_mid_body,
        out_shape=jax.ShapeDtypeStruct((N, W), jnp.float32),
    )(parts, recip, h, root_W, conv_b.reshape(1, W))


def _update_final(parts, recip, h, root_W, conv_b, fc2_W, fc2_b):
    return pl.pallas_call(
        _upd_fin_body,
        out_shape=jax.ShapeDtypeStruct((N, 1), jnp.float32),
    )(parts, recip, h, root_W, conv_b.reshape(1, W), fc2_W, fc2_b.reshape(1, 1))


@functools.partial(
    pl.kernel,
    out_type=jax.ShapeDtypeStruct((E, W), jnp.float32),
    mesh=_SC_MESH,
    scratch_types=[
        pltpu.VMEM((CPW, CH), jnp.int32),
        pltpu.VMEM((2, SUPE, W), jnp.float32),
        pltpu.SemaphoreType.DMA((2,)),
        pltpu.SemaphoreType.DMA((2,)),
    ],
    compiler_params=_SC_PARAMS,
)
def _sc_gather(h_hbm, src_hbm, out_hbm, idx_v, big_v, gsem, wsem):
    c = lax.axis_index("c")
    s = lax.axis_index("s")
    w = c * NS + s
    base = w * CPW
    pltpu.sync_copy(src_hbm.at[pl.ds(base, CPW)], idx_v)

    def fire_super(sidx, buf):
        for j in range(SUP):
            pltpu.async_copy(h_hbm.at[idx_v.at[sidx * SUP + j]],
                             big_v.at[buf, pl.ds(j * CH, CH)], gsem.at[buf])

    fire_super(0, 0)

    def body(sidx, _):
        buf = lax.rem(sidx, 2)
        obuf = 1 - buf

        @pl.when(sidx + 1 < NSUP)
        def _():
            @pl.when(sidx >= 1)
            def _():
                pltpu.make_async_copy(big_v.at[obuf],
                                      out_hbm.at[pl.ds(0, SUPE)],
                                      wsem.at[obuf]).wait()
            fire_super(sidx + 1, obuf)

        for j in range(SUP):
            pltpu.make_async_copy(h_hbm.at[idx_v.at[0]],
                                  big_v.at[buf, pl.ds(0, CH)],
                                  gsem.at[buf]).wait()
        pltpu.async_copy(big_v.at[buf],
                         out_hbm.at[pl.ds((base + sidx * SUP) * CH, SUPE)],
                         wsem.at[buf])
        return 0

    lax.fori_loop(0, NSUP, body, 0)
    for t in (NSUP - 2, NSUP - 1):
        pltpu.make_async_copy(big_v.at[t % 2], out_hbm.at[pl.ds(0, SUPE)],
                              wsem.at[t % 2]).wait()


def _make_sc_scatter(is_cnt):
    @functools.partial(
        pl.kernel,
        out_type=jax.ShapeDtypeStruct((NC * N, W), jnp.float32),
        mesh=_SC_MESH,
        scratch_types=[
            pltpu.VMEM_SHARED((N, W), jnp.float32),
            pltpu.VMEM((CPW, CH), jnp.int32),
            pltpu.VMEM((2, SUPE, W), jnp.float32),
            pltpu.SemaphoreType.DMA((2,)),
            pltpu.SemaphoreType.DMA((2,)),
        ],
        compiler_params=_SC_PARAMS,
    )
    def _sc_scatter(*args):
        if is_cnt:
            dst_hbm, zeros_hbm, ones_hbm, out_hbm, acc, idx_v, big_v, lsem, asem = args
        else:
            msg_hbm, dst_hbm, zeros_hbm, out_hbm, acc, idx_v, big_v, lsem, asem = args
        c = lax.axis_index("c")
        s = lax.axis_index("s")
        w = c * NS + s
        base = w * CPW
        pltpu.sync_copy(zeros_hbm.at[pl.ds(s * NPT, NPT)],
                        acc.at[pl.ds(s * NPT, NPT)])
        plsc.subcore_barrier()
        pltpu.sync_copy(dst_hbm.at[pl.ds(base, CPW)], idx_v)

        def fire_adds(sidx, buf):
            for j in range(SUP):
                pltpu.async_copy(big_v.at[buf, pl.ds(j * CH, CH)],
                                 acc.at[idx_v.at[sidx * SUP + j]],
                                 asem.at[buf], add=True)

        def drain_adds(buf):
            for j in range(SUP):
                pltpu.make_async_copy(big_v.at[buf, pl.ds(0, CH)],
                                      acc.at[idx_v.at[0]], asem.at[buf]).wait()

        if is_cnt:
            pltpu.sync_copy(ones_hbm, big_v.at[0, pl.ds(0, CH)])

            def body(sidx, _):
                for j in range(SUP):
                    pltpu.async_copy(big_v.at[0, pl.ds(0, CH)],
                                     acc.at[idx_v.at[sidx * SUP + j]],
                                     asem.at[0], add=True)
                drain_adds(0)
                return 0

            lax.fori_loop(0, NSUP, body, 0)
        else:
            pltpu.async_copy(msg_hbm.at[pl.ds(base * CH, SUPE)], big_v.at[0],
                             lsem.at[0])

            def body(sidx, _):
                buf = lax.rem(sidx, 2)
                obuf = 1 - buf
                pltpu.make_async_copy(msg_hbm.at[pl.ds(0, SUPE)],
                                      big_v.at[buf], lsem.at[buf]).wait()

                @pl.when(sidx + 1 < NSUP)
                def _():
                    @pl.when(sidx >= 1)
                    def _():
                        drain_adds(obuf)
                    pltpu.async_copy(
                        msg_hbm.at[pl.ds((base + (sidx + 1) * SUP) * CH, SUPE)],
                        big_v.at[obuf], lsem.at[obuf])

                fire_adds(sidx, buf)
                return 0

            lax.fori_loop(0, NSUP, body, 0)
            for t in (NSUP - 2, NSUP - 1):
                drain_adds(t % 2)

        plsc.subcore_barrier()
        pltpu.sync_copy(acc.at[pl.ds(s * NPT, NPT)],
                        out_hbm.at[pl.ds(c * N + s * NPT, NPT)])

    return _sc_scatter


_sc_scatter_msg = _make_sc_scatter(False)
_sc_scatter_cnt = _make_sc_scatter(True)


def kernel(x, edge_index, edge_attr, fc1_W, fc1_b, k1_W, k1_b, k2_W, k2_b,
           root_W, conv_b, fc2_W, fc2_b):
    srcp = edge_index[0].reshape(NCHUNK, CH)
    dstp = edge_index[1].reshape(NCHUNK, CH)
    kcat = jnp.concatenate([k2_W.reshape(8 * W, W), k2_b.reshape(W, W)], axis=0)
    irep = jnp.tile(jnp.eye(W, dtype=jnp.float32), (1, R))
    srep = jnp.repeat(jnp.eye(16, dtype=jnp.float32), W, axis=1)[:, :R * W]
    eye4 = jnp.eye(4, dtype=jnp.float32)
    sreps = jnp.stack([jnp.kron(eye4, srep[:16, r * W:(r + 1) * W])
                       for r in range(R)]).astype(jnp.bfloat16)
    kcats = jnp.stack([jnp.kron(eye4, kcat[r * W:(r + 1) * W, :])
                       for r in range(R)]).astype(jnp.bfloat16)
    k1x = jnp.pad(k1_W, ((0, 0), (0, 8)))
    k1_W4 = jnp.kron(eye4, k1x)
    k1_b4 = jnp.tile(jnp.pad(k1_b, (0, 8)), 4).reshape(1, 64)
    lane64 = jnp.arange(64) % 16
    ones_mask = (lane64 == 8).astype(jnp.float32).reshape(1, 64)
    zeros_nw = jnp.zeros((N, W), jnp.float32)
    ones_cw = jnp.ones((CH, W), jnp.float32)

    h = _h0(x, fc1_W, fc1_b)
    ew4 = _ew16(edge_attr.reshape(E // 4, 28), k1_W4, k1_b4, ones_mask)
    cnt_parts = _sc_scatter_cnt(dstp, zeros_nw, ones_cw)
    recip = _recip(cnt_parts)

    for layer in range(DEPTH):
        xj = _sc_gather(h, srcp)
        msg4 = _msg(xj.reshape(E // 4, 128), ew4, sreps, kcats)
        parts = _sc_scatter_msg(msg4.reshape(E, W), dstp, zeros_nw)
        if layer < DEPTH - 1:
            h = _update(parts, recip, h, root_W, conv_b)
        else:
            out = _update_final(parts, recip, h, root_W, conv_b, fc2_W, fc2_b)
    return out


# speedup vs baseline: 6.4801x; 1.0703x over previous
import functools

import jax
import jax.numpy as jnp
from jax import lax
from jax.experimental import pallas as pl
from jax.experimental.pallas import tpu as pltpu
from jax.experimental.pallas import tpu_sc as plsc

N = 10000
E = 160000
W = 32
DEPTH = 4
R = 9
NC, NS = 2, 16
NW = NC * NS
CH = 125
NCHUNK = E // CH
CPW = NCHUNK // NW
SUP = 8
NSUP = CPW // SUP
SUPE = SUP * CH
NPT = N // NS
BE = 8000

_SC_MESH = plsc.VectorSubcoreMesh(
    core_axis_name="c", subcore_axis_name="s", num_cores=NC, num_subcores=NS)
_SC_PARAMS = pltpu.CompilerParams(use_tc_tiling_on_sc=False)


def _h0_body(x_ref, w_ref, b_ref, o_ref):
    o_ref[...] = jnp.dot(x_ref[...], w_ref[...],
                         preferred_element_type=jnp.float32) + b_ref[...]


def _h0(x, fc1_W, fc1_b):
    return pl.pallas_call(
        _h0_body,
        out_shape=jax.ShapeDtypeStruct((N, W), jnp.float32),
    )(x, fc1_W, fc1_b.reshape(1, W))


def _ew_body(ea_ref, w_ref, b_ref, m_ref, o_ref):
    u = jnp.dot(ea_ref[...], w_ref[...],
                preferred_element_type=jnp.float32) + b_ref[...]
    o_ref[...] = jax.nn.relu(u) + m_ref[...]


def _ew16(ea4, k1_W4, k1_b4, ones_mask):
    b4 = BE // 4
    return pl.pallas_call(
        _ew_body,
        grid=(E // BE,),
        in_specs=[pl.BlockSpec((b4, 28), lambda i: (i, 0)),
                  pl.BlockSpec((28, 64), lambda i: (0, 0)),
                  pl.BlockSpec((1, 64), lambda i: (0, 0)),
                  pl.BlockSpec((1, 64), lambda i: (0, 0))],
        out_specs=pl.BlockSpec((b4, 64), lambda i: (i, 0)),
        out_shape=jax.ShapeDtypeStruct((E // 4, 64), jnp.float32),
    )(ea4, k1_W4, k1_b4, ones_mask)


def _msg_body(xj_ref, ew_ref, s_ref, k_ref, o_ref):
    bf = jnp.bfloat16
    xj = xj_ref[...].astype(bf)
    ew = ew_ref[...].astype(bf)
    acc = jnp.zeros((BE // 4, 128), jnp.float32)
    for r in range(R):
        er = jnp.dot(ew, s_ref[r],
                     preferred_element_type=jnp.float32).astype(bf)
        acc = acc + jnp.dot(xj * er, k_ref[r],
                            preferred_element_type=jnp.float32)
    o_ref[...] = acc


def _msg(xj4, ew4, sreps, kcats, half):
    b4 = BE // 4
    eh = E // 2
    off = half * (eh // BE)
    return pl.pallas_call(
        _msg_body,
        grid=(eh // BE,),
        in_specs=[pl.BlockSpec((b4, 128), lambda i: (i, 0)),
                  pl.BlockSpec((b4, 64), lambda i: (i + off, 0)),
                  pl.BlockSpec((R, 64, 128), lambda i: (0, 0, 0)),
                  pl.BlockSpec((R, 128, 128), lambda i: (0, 0, 0))],
        out_specs=pl.BlockSpec((b4, 128), lambda i: (i, 0)),
        out_shape=jax.ShapeDtypeStruct((eh // 4, 128), jnp.float32),
    )(xj4, ew4, sreps, kcats)


def _recip_body(p_ref, o_ref):
    cnt = p_ref[pl.ds(0, N), :] + p_ref[pl.ds(N, N), :]
    o_ref[...] = 1.0 / jnp.maximum(cnt, 1.0)


def _recip(parts):
    return pl.pallas_call(
        _recip_body,
        out_shape=jax.ShapeDtypeStruct((N, W), jnp.float32),
    )(parts)


def _update_body(p_ref, r_ref, h_ref, w_ref, b_ref, o_ref, *, final, fw_ref=None,
                 fb_ref=None):
    sums = p_ref[pl.ds(0, N), :] + p_ref[pl.ds(N, N), :]
    u = jax.nn.relu(sums * r_ref[...]
                    + jnp.dot(h_ref[...], w_ref[...],
                              preferred_element_type=jnp.float32)
                    + b_ref[...])
    if final:
        o_ref[...] = jnp.dot(u, fw_ref[...],
                             preferred_element_type=jnp.float32) + fb_ref[...]
    else:
        o_ref[...] = u


def _upd_mid_body(p_ref, r_ref, h_ref, w_ref, b_ref, o_ref):
    _update_body(p_ref, r_ref, h_ref, w_ref, b_ref, o_ref, final=False)


def _upd_fin_body(p_ref, r_ref, h_ref, w_ref, b_ref, fw_ref, fb_ref, o_ref):
    _update_body(p_ref, r_ref, h_ref, w_ref, b_ref, o_ref, final=True,
                 fw_ref=fw_ref, fb_ref=fb_ref)


def _update(parts, recip, h, root_W, conv_b):
    return pl.pallas_call(
        _upd_mid_body,
        out_shape=jax.ShapeDtypeStruct((N, W), jnp.float32),
    )(parts, recip, h, root_W, conv_b.reshape(1, W))


def _update_final(parts, recip, h, root_W, conv_b, fc2_W, fc2_b):
    return pl.pallas_call(
        _upd_fin_body,
        out_shape=jax.ShapeDtypeStruct((N, 1), jnp.float32),
    )(parts, recip, h, root_W, conv_b.reshape(1, W), fc2_W, fc2_b.reshape(1, 1))


def _make_sc_gather(off, nch, sup):
    cpw = nch // NW
    nsup = cpw // sup
    supe = sup * CH

    @functools.partial(
        pl.kernel,
        out_type=jax.ShapeDtypeStruct((nch * CH, W), jnp.float32),
        mesh=_SC_MESH,
        scratch_types=[
            pltpu.VMEM((cpw, CH), jnp.int32),
            pltpu.VMEM((2, supe, W), jnp.float32),
            pltpu.SemaphoreType.DMA((2,)),
            pltpu.SemaphoreType.DMA((2,)),
        ],
        compiler_params=_SC_PARAMS,
    )
    def _sc_gather(h_hbm, src_hbm, out_hbm, idx_v, big_v, gsem, wsem):
        c = lax.axis_index("c")
        s = lax.axis_index("s")
        w = c * NS + s
        base = w * cpw
        pltpu.sync_copy(src_hbm.at[pl.ds(off + base, cpw)], idx_v)

        def fire_super(sidx, buf):
            for j in range(sup):
                pltpu.async_copy(h_hbm.at[idx_v.at[sidx * sup + j]],
                                 big_v.at[buf, pl.ds(j * CH, CH)], gsem.at[buf])

        fire_super(0, 0)

        def body(sidx, _):
            buf = lax.rem(sidx, 2)
            obuf = 1 - buf

            @pl.when(sidx + 1 < nsup)
            def _():
                @pl.when(sidx >= 1)
                def _():
                    pltpu.make_async_copy(big_v.at[obuf],
                                          out_hbm.at[pl.ds(0, supe)],
                                          wsem.at[obuf]).wait()
                fire_super(sidx + 1, obuf)

            for j in range(sup):
                pltpu.make_async_copy(h_hbm.at[idx_v.at[0]],
                                      big_v.at[buf, pl.ds(0, CH)],
                                      gsem.at[buf]).wait()
            pltpu.async_copy(big_v.at[buf],
                             out_hbm.at[pl.ds((base + sidx * sup) * CH, supe)],
                             wsem.at[buf])
            return 0

        lax.fori_loop(0, nsup, body, 0)
        for t in (nsup - 2, nsup - 1):
            pltpu.make_async_copy(big_v.at[t % 2], out_hbm.at[pl.ds(0, supe)],
                                  wsem.at[t % 2]).wait()

    return _sc_gather


def _make_sc_scatter(is_cnt, off, nch, sup):
    cpw = nch // NW
    nsup = cpw // sup
    supe = sup * CH

    @functools.partial(
        pl.kernel,
        out_type=jax.ShapeDtypeStruct((NC * N, W), jnp.float32),
        mesh=_SC_MESH,
        scratch_types=[
            pltpu.VMEM_SHARED((N, W), jnp.float32),
            pltpu.VMEM((cpw, CH), jnp.int32),
            pltpu.VMEM((2, supe, W), jnp.float32),
            pltpu.SemaphoreType.DMA((2,)),
            pltpu.SemaphoreType.DMA((2,)),
        ],
        compiler_params=_SC_PARAMS,
    )
    def _sc_scatter(*args):
        if is_cnt:
            dst_hbm, init_hbm, ones_hbm, out_hbm, acc, idx_v, big_v, lsem, asem = args
        else:
            msg_hbm, dst_hbm, init_hbm, out_hbm, acc, idx_v, big_v, lsem, asem = args
        c = lax.axis_index("c")
        s = lax.axis_index("s")
        w = c * NS + s
        base = w * cpw
        pltpu.sync_copy(init_hbm.at[pl.ds(c * N + s * NPT, NPT)],
                        acc.at[pl.ds(s * NPT, NPT)])
        plsc.subcore_barrier()
        pltpu.sync_copy(dst_hbm.at[pl.ds(off + base, cpw)], idx_v)

        def fire_adds(sidx, buf):
            for j in range(sup):
                pltpu.async_copy(big_v.at[buf, pl.ds(j * CH, CH)],
                                 acc.at[idx_v.at[sidx * sup + j]],
                                 asem.at[buf], add=True)

        def drain_adds(buf):
            for j in range(sup):
                pltpu.make_async_copy(big_v.at[buf, pl.ds(0, CH)],
                                      acc.at[idx_v.at[0]], asem.at[buf]).wait()

        if is_cnt:
            pltpu.sync_copy(ones_hbm, big_v.at[0, pl.ds(0, CH)])

            def body(sidx, _):
                for j in range(sup):
                    pltpu.async_copy(big_v.at[0, pl.ds(0, CH)],
                                     acc.at[idx_v.at[sidx * sup + j]],
                                     asem.at[0], add=True)
                drain_adds(0)
                return 0

            lax.fori_loop(0, nsup, body, 0)
        else:
            pltpu.async_copy(msg_hbm.at[pl.ds(base * CH, supe)], big_v.at[0],
                             lsem.at[0])

            def body(sidx, _):
                buf = lax.rem(sidx, 2)
                obuf = 1 - buf
                pltpu.make_async_copy(msg_hbm.at[pl.ds(0, supe)],
                                      big_v.at[buf], lsem.at[buf]).wait()

                @pl.when(sidx + 1 < nsup)
                def _():
                    @pl.when(sidx >= 1)
                    def _():
                        drain_adds(obuf)
                    pltpu.async_copy(
                        msg_hbm.at[pl.ds((base + (sidx + 1) * sup) * CH, supe)],
                        big_v.at[obuf], lsem.at[obuf])

                fire_adds(sidx, buf)
                return 0

            lax.fori_loop(0, nsup, body, 0)
            for t in (nsup - 2, nsup - 1):
                drain_adds(t % 2)

        plsc.subcore_barrier()
        pltpu.sync_copy(acc.at[pl.ds(s * NPT, NPT)],
                        out_hbm.at[pl.ds(c * N + s * NPT, NPT)])

    return _sc_scatter


NCHH = NCHUNK // 2
EH = NCHH * CH
_sc_gather_a = _make_sc_gather(0, NCHH, 10)
_sc_gather_b = _make_sc_gather(NCHH, NCHH, 10)
_sc_scatter_a = _make_sc_scatter(False, 0, NCHH, 10)
_sc_scatter_b = _make_sc_scatter(False, NCHH, NCHH, 10)
_sc_scatter_cnt = _make_sc_scatter(True, 0, NCHUNK, SUP)


def kernel(x, edge_index, edge_attr, fc1_W, fc1_b, k1_W, k1_b, k2_W, k2_b,
           root_W, conv_b, fc2_W, fc2_b):
    srcp = edge_index[0].reshape(NCHUNK, CH)
    dstp = edge_index[1].reshape(NCHUNK, CH)
    kcat = jnp.concatenate([k2_W.reshape(8 * W, W), k2_b.reshape(W, W)], axis=0)
    irep = jnp.tile(jnp.eye(W, dtype=jnp.float32), (1, R))
    srep = jnp.repeat(jnp.eye(16, dtype=jnp.float32), W, axis=1)[:, :R * W]
    eye4 = jnp.eye(4, dtype=jnp.float32)
    sreps = jnp.stack([jnp.kron(eye4, srep[:16, r * W:(r + 1) * W])
                       for r in range(R)]).astype(jnp.bfloat16)
    kcats = jnp.stack([jnp.kron(eye4, kcat[r * W:(r + 1) * W, :])
                       for r in range(R)]).astype(jnp.bfloat16)
    k1x = jnp.pad(k1_W, ((0, 0), (0, 8)))
    k1_W4 = jnp.kron(eye4, k1x)
    k1_b4 = jnp.tile(jnp.pad(k1_b, (0, 8)), 4).reshape(1, 64)
    lane64 = jnp.arange(64) % 16
    ones_mask = (lane64 == 8).astype(jnp.float32).reshape(1, 64)
    zeros2 = jnp.zeros((NC * N, W), jnp.float32)
    ones_cw = jnp.ones((CH, W), jnp.float32)

    h = _h0(x, fc1_W, fc1_b)
    ew4 = _ew16(edge_attr.reshape(E // 4, 28), k1_W4, k1_b4, ones_mask)
    cnt_parts = _sc_scatter_cnt(dstp, zeros2, ones_cw)
    recip = _recip(cnt_parts)

    for layer in range(DEPTH):
        xja = _sc_gather_a(h, srcp)
        xjb = _sc_gather_b(h, srcp)
        msga = _msg(xja.reshape(EH // 4, 128), ew4, sreps, kcats, 0)
        msgb = _msg(xjb.reshape(EH // 4, 128), ew4, sreps, kcats, 1)
        pa = _sc_scatter_a(msga.reshape(EH, W), dstp, zeros2)
        parts = _sc_scatter_b(msgb.reshape(EH, W), dstp, pa)
        if layer < DEPTH - 1:
            h = _update(parts, recip, h, root_W, conv_b)
        else:
            out = _update_final(parts, recip, h, root_W, conv_b, fc2_W, fc2_b)
    return out


# speedup vs baseline: 7.2481x; 1.1185x over previous
import functools

import jax
import jax.numpy as jnp
from jax import lax
from jax.experimental import pallas as pl
from jax.experimental.pallas import tpu as pltpu
from jax.experimental.pallas import tpu_sc as plsc

N = 10000
E = 160000
W = 32
DEPTH = 4
R = 9
NC, NS = 2, 16
NW = NC * NS
CH = 125
NCHUNK = E // CH
CPW = NCHUNK // NW
SUP = 8
NSUP = CPW // SUP
SUPE = SUP * CH
NPT = N // NS
BE = 8000

_SC_MESH = plsc.VectorSubcoreMesh(
    core_axis_name="c", subcore_axis_name="s", num_cores=NC, num_subcores=NS)
_SC_PARAMS = pltpu.CompilerParams(use_tc_tiling_on_sc=False)


def _h0_body(x_ref, w_ref, b_ref, o_ref):
    o_ref[...] = jnp.dot(x_ref[...], w_ref[...],
                         preferred_element_type=jnp.float32) + b_ref[...]


def _h0(x, fc1_W, fc1_b):
    return pl.pallas_call(
        _h0_body,
        out_shape=jax.ShapeDtypeStruct((N, W), jnp.float32),
    )(x, fc1_W, fc1_b.reshape(1, W))


def _ew_body(ea_ref, w_ref, b_ref, m_ref, o_ref):
    u = jnp.dot(ea_ref[...], w_ref[...],
                preferred_element_type=jnp.float32) + b_ref[...]
    o_ref[...] = jax.nn.relu(u) + m_ref[...]


def _ew16(ea4, k1_W4, k1_b4, ones_mask):
    b4 = BE // 4
    return pl.pallas_call(
        _ew_body,
        grid=(E // BE,),
        in_specs=[pl.BlockSpec((b4, 28), lambda i: (i, 0)),
                  pl.BlockSpec((28, 64), lambda i: (0, 0)),
                  pl.BlockSpec((1, 64), lambda i: (0, 0)),
                  pl.BlockSpec((1, 64), lambda i: (0, 0))],
        out_specs=pl.BlockSpec((b4, 64), lambda i: (i, 0)),
        out_shape=jax.ShapeDtypeStruct((E // 4, 64), jnp.float32),
    )(ea4, k1_W4, k1_b4, ones_mask)


def _msg_body(xj_ref, ew_ref, s_ref, k_ref, o_ref):
    bf = jnp.bfloat16
    xj = xj_ref[...].astype(bf)
    ew = ew_ref[...].astype(bf)
    er_all = jnp.dot(ew, s_ref[...],
                     preferred_element_type=jnp.float32).astype(bf)
    acc = jnp.zeros((BE // 4, 128), jnp.float32)
    for r in range(R):
        er = er_all[:, r * 128:(r + 1) * 128]
        acc = acc + jnp.dot(xj * er, k_ref[r],
                            preferred_element_type=jnp.float32)
    o_ref[...] = acc


def _msg(xj4, ew4, sreps, kcats, half):
    b4 = BE // 4
    eh = E // 2
    off = half * (eh // BE)
    return pl.pallas_call(
        _msg_body,
        grid=(eh // BE,),
        in_specs=[pl.BlockSpec((b4, 128), lambda i: (i, 0)),
                  pl.BlockSpec((b4, 64), lambda i: (i + off, 0)),
                  pl.BlockSpec((64, R * 128), lambda i: (0, 0)),
                  pl.BlockSpec((R, 128, 128), lambda i: (0, 0, 0))],
        out_specs=pl.BlockSpec((b4, 128), lambda i: (i, 0)),
        out_shape=jax.ShapeDtypeStruct((eh // 4, 128), jnp.float32),
    )(xj4, ew4, sreps, kcats)


def _recip_body(p_ref, o_ref):
    cnt = p_ref[pl.ds(0, N), :] + p_ref[pl.ds(N, N), :]
    o_ref[...] = 1.0 / jnp.maximum(cnt, 1.0)


def _recip(parts):
    return pl.pallas_call(
        _recip_body,
        out_shape=jax.ShapeDtypeStruct((N, W), jnp.float32),
    )(parts)


def _update_body(p_ref, r_ref, h_ref, w_ref, b_ref, o_ref, *, final, fw_ref=None,
                 fb_ref=None):
    sums = p_ref[pl.ds(0, N), :] + p_ref[pl.ds(N, N), :]
    u = jax.nn.relu(sums * r_ref[...]
                    + jnp.dot(h_ref[...], w_ref[...],
                              preferred_element_type=jnp.float32)
                    + b_ref[...])
    if final:
        o_ref[...] = jnp.dot(u, fw_ref[...],
                             preferred_element_type=jnp.float32) + fb_ref[...]
    else:
        o_ref[...] = u


def _upd_mid_body(p_ref, r_ref, h_ref, w_ref, b_ref, o_ref):
    _update_body(p_ref, r_ref, h_ref, w_ref, b_ref, o_ref, final=False)


def _upd_fin_body(p_ref, r_ref, h_ref, w_ref, b_ref, fw_ref, fb_ref, o_ref):
    _update_body(p_ref, r_ref, h_ref, w_ref, b_ref, o_ref, final=True,
                 fw_ref=fw_ref, fb_ref=fb_ref)


def _update(parts, recip, h, root_W, conv_b):
    return pl.pallas_call(
        _upd_mid_body,
        out_shape=jax.ShapeDtypeStruct((N, W), jnp.float32),
    )(parts, recip, h, root_W, conv_b.reshape(1, W))


def _update_final(parts, recip, h, root_W, conv_b, fc2_W, fc2_b):
    return pl.pallas_call(
        _upd_fin_body,
        out_shape=jax.ShapeDtypeStruct((N, 1), jnp.float32),
    )(parts, recip, h, root_W, conv_b.reshape(1, W), fc2_W, fc2_b.reshape(1, 1))


def _make_sc_gather(off, nch, sup):
    cpw = nch // NW
    nsup = cpw // sup
    supe = sup * CH

    @functools.partial(
        pl.kernel,
        out_type=jax.ShapeDtypeStruct((nch * CH, W), jnp.float32),
        mesh=_SC_MESH,
        scratch_types=[
            pltpu.VMEM((cpw, CH), jnp.int32),
            pltpu.VMEM((2, supe, W), jnp.float32),
            pltpu.SemaphoreType.DMA((2,)),
            pltpu.SemaphoreType.DMA((2,)),
        ],
        compiler_params=_SC_PARAMS,
    )
    def _sc_gather(h_hbm, src_hbm, out_hbm, idx_v, big_v, gsem, wsem):
        c = lax.axis_index("c")
        s = lax.axis_index("s")
        w = c * NS + s
        base = w * cpw
        pltpu.sync_copy(src_hbm.at[pl.ds(off + base, cpw)], idx_v)

        def fire_super(sidx, buf):
            for j in range(sup):
                pltpu.async_copy(h_hbm.at[idx_v.at[sidx * sup + j]],
                                 big_v.at[buf, pl.ds(j * CH, CH)], gsem.at[buf])

        fire_super(0, 0)

        def body(sidx, _):
            buf = lax.rem(sidx, 2)
            obuf = 1 - buf

            @pl.when(sidx + 1 < nsup)
            def _():
                @pl.when(sidx >= 1)
                def _():
                    pltpu.make_async_copy(big_v.at[obuf],
                                          out_hbm.at[pl.ds(0, supe)],
                                          wsem.at[obuf]).wait()
                fire_super(sidx + 1, obuf)

            for j in range(sup):
                pltpu.make_async_copy(h_hbm.at[idx_v.at[0]],
                                      big_v.at[buf, pl.ds(0, CH)],
                                      gsem.at[buf]).wait()
            pltpu.async_copy(big_v.at[buf],
                             out_hbm.at[pl.ds((base + sidx * sup) * CH, supe)],
                             wsem.at[buf])
            return 0

        lax.fori_loop(0, nsup, body, 0)
        for t in (nsup - 2, nsup - 1):
            pltpu.make_async_copy(big_v.at[t % 2], out_hbm.at[pl.ds(0, supe)],
                                  wsem.at[t % 2]).wait()

    return _sc_gather


def _make_sc_scatter(is_cnt, off, nch, sup):
    cpw = nch // NW
    nsup = cpw // sup
    supe = sup * CH

    @functools.partial(
        pl.kernel,
        out_type=jax.ShapeDtypeStruct((NC * N, W), jnp.float32),
        mesh=_SC_MESH,
        scratch_types=[
            pltpu.VMEM_SHARED((N, W), jnp.float32),
            pltpu.VMEM((cpw, CH), jnp.int32),
            pltpu.VMEM((2, supe, W), jnp.float32),
            pltpu.SemaphoreType.DMA((2,)),
            pltpu.SemaphoreType.DMA((2,)),
        ],
        compiler_params=_SC_PARAMS,
    )
    def _sc_scatter(*args):
        if is_cnt:
            dst_hbm, init_hbm, ones_hbm, out_hbm, acc, idx_v, big_v, lsem, asem = args
        else:
            msg_hbm, dst_hbm, init_hbm, out_hbm, acc, idx_v, big_v, lsem, asem = args
        c = lax.axis_index("c")
        s = lax.axis_index("s")
        w = c * NS + s
        base = w * cpw
        pltpu.sync_copy(init_hbm.at[pl.ds(c * N + s * NPT, NPT)],
                        acc.at[pl.ds(s * NPT, NPT)])
        plsc.subcore_barrier()
        pltpu.sync_copy(dst_hbm.at[pl.ds(off + base, cpw)], idx_v)

        def fire_adds(sidx, buf):
            for j in range(sup):
                pltpu.async_copy(big_v.at[buf, pl.ds(j * CH, CH)],
                                 acc.at[idx_v.at[sidx * sup + j]],
                                 asem.at[buf], add=True)

        def drain_adds(buf):
            for j in range(sup):
                pltpu.make_async_copy(big_v.at[buf, pl.ds(0, CH)],
                                      acc.at[idx_v.at[0]], asem.at[buf]).wait()

        if is_cnt:
            pltpu.sync_copy(ones_hbm, big_v.at[0, pl.ds(0, CH)])

            def body(sidx, _):
                for j in range(sup):
                    pltpu.async_copy(big_v.at[0, pl.ds(0, CH)],
                                     acc.at[idx_v.at[sidx * sup + j]],
                                     asem.at[0], add=True)
                drain_adds(0)
                return 0

            lax.fori_loop(0, nsup, body, 0)
        else:
            pltpu.async_copy(msg_hbm.at[pl.ds(base * CH, supe)], big_v.at[0],
                             lsem.at[0])

            def body(sidx, _):
                buf = lax.rem(sidx, 2)
                obuf = 1 - buf
                pltpu.make_async_copy(msg_hbm.at[pl.ds(0, supe)],
                                      big_v.at[buf], lsem.at[buf]).wait()

                @pl.when(sidx + 1 < nsup)
                def _():
                    @pl.when(sidx >= 1)
                    def _():
                        drain_adds(obuf)
                    pltpu.async_copy(
                        msg_hbm.at[pl.ds((base + (sidx + 1) * sup) * CH, supe)],
                        big_v.at[obuf], lsem.at[obuf])

                fire_adds(sidx, buf)
                return 0

            lax.fori_loop(0, nsup, body, 0)
            for t in (nsup - 2, nsup - 1):
                drain_adds(t % 2)

        plsc.subcore_barrier()
        pltpu.sync_copy(acc.at[pl.ds(s * NPT, NPT)],
                        out_hbm.at[pl.ds(c * N + s * NPT, NPT)])

    return _sc_scatter


NCHH = NCHUNK // 2
EH = NCHH * CH
_sc_gather_a = _make_sc_gather(0, NCHH, 10)
_sc_gather_b = _make_sc_gather(NCHH, NCHH, 10)
_sc_scatter_a = _make_sc_scatter(False, 0, NCHH, 10)
_sc_scatter_b = _make_sc_scatter(False, NCHH, NCHH, 10)
_sc_scatter_cnt = _make_sc_scatter(True, 0, NCHUNK, SUP)


def kernel(x, edge_index, edge_attr, fc1_W, fc1_b, k1_W, k1_b, k2_W, k2_b,
           root_W, conv_b, fc2_W, fc2_b):
    srcp = edge_index[0].reshape(NCHUNK, CH)
    dstp = edge_index[1].reshape(NCHUNK, CH)
    kcat = jnp.concatenate([k2_W.reshape(8 * W, W), k2_b.reshape(W, W)], axis=0)
    irep = jnp.tile(jnp.eye(W, dtype=jnp.float32), (1, R))
    srep = jnp.repeat(jnp.eye(16, dtype=jnp.float32), W, axis=1)[:, :R * W]
    eye4 = jnp.eye(4, dtype=jnp.float32)
    sreps = jnp.concatenate([jnp.kron(eye4, srep[:16, r * W:(r + 1) * W])
                             for r in range(R)], axis=1).astype(jnp.bfloat16)
    kcats = jnp.stack([jnp.kron(eye4, kcat[r * W:(r + 1) * W, :])
                       for r in range(R)]).astype(jnp.bfloat16)
    k1x = jnp.pad(k1_W, ((0, 0), (0, 8)))
    k1_W4 = jnp.kron(eye4, k1x)
    k1_b4 = jnp.tile(jnp.pad(k1_b, (0, 8)), 4).reshape(1, 64)
    lane64 = jnp.arange(64) % 16
    ones_mask = (lane64 == 8).astype(jnp.float32).reshape(1, 64)
    zeros2 = jnp.zeros((NC * N, W), jnp.float32)
    ones_cw = jnp.ones((CH, W), jnp.float32)

    h = _h0(x, fc1_W, fc1_b)
    ew4 = _ew16(edge_attr.reshape(E // 4, 28), k1_W4, k1_b4, ones_mask)
    cnt_parts = _sc_scatter_cnt(dstp, zeros2, ones_cw)
    recip = _recip(cnt_parts)

    for layer in range(DEPTH):
        xja = _sc_gather_a(h, srcp)
        xjb = _sc_gather_b(h, srcp)
        msga = _msg(xja.reshape(EH // 4, 128), ew4, sreps, kcats, 0)
        msgb = _msg(xjb.reshape(EH // 4, 128), ew4, sreps, kcats, 1)
        pa = _sc_scatter_a(msga.reshape(EH, W), dstp, zeros2)
        parts = _sc_scatter_b(msgb.reshape(EH, W), dstp, pa)
        if layer < DEPTH - 1:
            h = _update(parts, recip, h, root_W, conv_b)
        else:
            out = _update_final(parts, recip, h, root_W, conv_b, fc2_W, fc2_b)
    return out
